# baseline (speedup 1.0000x reference)
import functools

import jax
import jax.numpy as jnp
from jax import lax
from jax.experimental import pallas as pl
from jax.experimental.pallas import tpu as pltpu
from jax.experimental.pallas import tpu_sc as plsc

BATCH = 16384
FEAT = 2048
L = 16
NC, NS = 2, 16
NW = NC * NS
ROWS_PER_W = BATCH // NW
IBLK = 8
OBLK = 16
NBLK = ROWS_PER_W // OBLK
NJ = FEAT // L

_mesh = plsc.VectorSubcoreMesh(core_axis_name="c", subcore_axis_name="s")


@functools.partial(
    pl.kernel,
    mesh=_mesh,
    compiler_params=pltpu.CompilerParams(needs_layout_passes=False),
    out_type=jax.ShapeDtypeStruct((BATCH, FEAT), jnp.float32),
    scratch_types=[
        pltpu.VMEM((FEAT,), jnp.int32),
        pltpu.VMEM((IBLK, FEAT), jnp.float32),
        pltpu.VMEM((IBLK, FEAT), jnp.float32),
        pltpu.VMEM((OBLK, FEAT), jnp.float32),
        pltpu.VMEM((OBLK, FEAT), jnp.float32),
        pltpu.SemaphoreType.DMA,
        pltpu.SemaphoreType.DMA,
        pltpu.SemaphoreType.DMA,
        pltpu.SemaphoreType.DMA,
        pltpu.SemaphoreType.DMA,
    ],
)
def _permute_sc(in_hbm, p_hbm, out_hbm, p_v, in0, in1, out0, out1,
                isem0, isem1, osem0, osem1, psem):
    wid = lax.axis_index("s") * NC + lax.axis_index("c")
    base = wid * ROWS_PER_W

    ins = (in0, in1)
    outs = (out0, out1)
    isems = (isem0, isem1)
    osems = (osem0, osem1)

    def in_copy(k, s):
        return pltpu.make_async_copy(
            in_hbm.at[pl.ds(base + k * IBLK, IBLK)], ins[s], isems[s])

    def out_copy(b, o):
        return pltpu.make_async_copy(
            outs[o], out_hbm.at[pl.ds(base + b * OBLK, OBLK)], osems[o])

    def permute_half(in_ref, out_ref, h):
        @plsc.parallel_loop(0, NJ, unroll=4)
        def _groups(j):
            pj = p_v[pl.ds(j * L, L)]
            for r in range(IBLK):
                rows = jnp.full((L,), r, jnp.int32)
                vals = plsc.load_gather(in_ref, [rows, pj])
                out_ref[h * IBLK + r, pl.ds(j * L, L)] = vals

    p_dma = pltpu.make_async_copy(p_hbm, p_v, psem)
    p_dma.start()
    in_copy(0, 0).start()
    in_copy(1, 1).start()
    p_dma.wait()

    @pl.loop(0, NBLK, step=2)
    def _blocks(bb):
        for o in range(2):
            b = bb + o

            @pl.when(b >= 2)
            def _():
                out_copy(b - 2, o).wait()

            for h in range(2):
                s = (2 * o + h) % 2
                k = 2 * b + h
                in_copy(k, s).wait()
                permute_half(ins[s], outs[o], h)

                @pl.when(k + 2 < 2 * NBLK)
                def _():
                    in_copy(k + 2, s).start()

            out_copy(b, o).start()

    out_copy(NBLK - 2, 0).wait()
    out_copy(NBLK - 1, 1).wait()


def kernel(inputs, p):
    return _permute_sc(inputs, p)


# speedup vs baseline: 1.0411x; 1.0411x over previous
import functools

import jax
import jax.numpy as jnp
from jax import lax
from jax.experimental import pallas as pl
from jax.experimental.pallas import tpu as pltpu
from jax.experimental.pallas import tpu_sc as plsc

BATCH = 16384
FEAT = 2048
L = 16
NC, NS = 2, 16
NW = NC * NS
ROWS_PER_W = BATCH // NW
BLK = 8
NBLK = ROWS_PER_W // BLK
NIN = 4
NOUT = 2
NJ = FEAT // L

_mesh = plsc.VectorSubcoreMesh(core_axis_name="c", subcore_axis_name="s")


@functools.partial(
    pl.kernel,
    mesh=_mesh,
    compiler_params=pltpu.CompilerParams(needs_layout_passes=False),
    out_type=jax.ShapeDtypeStruct((BATCH, FEAT), jnp.float32),
    scratch_types=[
        pltpu.VMEM((FEAT,), jnp.int32),
        pltpu.VMEM((BLK, FEAT), jnp.float32),
        pltpu.VMEM((BLK, FEAT), jnp.float32),
        pltpu.VMEM((BLK, FEAT), jnp.float32),
        pltpu.VMEM((BLK, FEAT), jnp.float32),
        pltpu.VMEM((BLK, FEAT), jnp.float32),
        pltpu.VMEM((BLK, FEAT), jnp.float32),
        pltpu.SemaphoreType.DMA,
        pltpu.SemaphoreType.DMA,
        pltpu.SemaphoreType.DMA,
        pltpu.SemaphoreType.DMA,
        pltpu.SemaphoreType.DMA,
        pltpu.SemaphoreType.DMA,
        pltpu.SemaphoreType.DMA,
    ],
)
def _permute_sc(in_hbm, p_hbm, out_hbm, p_v, in0, in1, in2, in3, out0, out1,
                isem0, isem1, isem2, isem3, osem0, osem1, psem):
    wid = lax.axis_index("s") * NC + lax.axis_index("c")
    base = wid * ROWS_PER_W

    ins = (in0, in1, in2, in3)
    outs = (out0, out1)
    isems = (isem0, isem1, isem2, isem3)
    osems = (osem0, osem1)

    def in_copy(b, s):
        return pltpu.make_async_copy(
            in_hbm.at[pl.ds(base + b * BLK, BLK)], ins[s], isems[s])

    def out_copy(b, o):
        return pltpu.make_async_copy(
            outs[o], out_hbm.at[pl.ds(base + b * BLK, BLK)], osems[o])

    def permute_block(in_ref, out_ref):
        @plsc.parallel_loop(0, NJ, unroll=4)
        def _groups(j):
            pj = p_v[pl.ds(j * L, L)]
            for r in range(BLK):
                rows = jnp.full((L,), r, jnp.int32)
                vals = plsc.load_gather(in_ref, [rows, pj])
                out_ref[r, pl.ds(j * L, L)] = vals

    p_dma = pltpu.make_async_copy(p_hbm, p_v, psem)
    p_dma.start()
    for k in range(NIN - 1):
        in_copy(k, k).start()
    p_dma.wait()

    @pl.loop(0, NBLK, step=NIN)
    def _blocks(bb):
        for s in range(NIN):
            b = bb + s

            @pl.when(b + NIN - 1 < NBLK)
            def _():
                in_copy(b + NIN - 1, (s + NIN - 1) % NIN).start()

            in_copy(b, s).wait()

            o = s % NOUT

            @pl.when(b >= NOUT)
            def _():
                out_copy(b - NOUT, o).wait()

            permute_block(ins[s], outs[o])
            out_copy(b, o).start()

    out_copy(NBLK - 2, 0).wait()
    out_copy(NBLK - 1, 1).wait()


def kernel(inputs, p):
    return _permute_sc(inputs, p)
